# baseline (speedup 1.0000x reference)
import functools

import jax
import jax.numpy as jnp
from jax import lax
from jax.experimental import pallas as pl
from jax.experimental.pallas import tpu as pltpu
from jax.experimental.pallas import tpu_sc as plsc

N = 10000
E = 320000
D = 128
H1 = 32
H2 = 2
H2P = 8

NC = 2
NS = 16
NW = NC * NS
LANES = 16

C = 512
EPW_RAW = E // NW
CPT = 20
EPT = CPT * C
NB = 5
LA = 3
SL = 2

NP = 10016
NPT = N // NS

DEG_S = NS * 640
HC = 512
HCT = E // HC
HFULL = HCT // NW
HEXTRA = HCT - HFULL * NW

f32 = jnp.float32
i32 = jnp.int32

_mesh = plsc.VectorSubcoreMesh(core_axis_name="c", subcore_axis_name="s")
_sc_params = pltpu.CompilerParams(use_tc_tiling_on_sc=False)


@functools.partial(
    pl.kernel,
    out_type=jax.ShapeDtypeStruct((NC, DEG_S), f32),
    mesh=_mesh,
    compiler_params=_sc_params,
    scratch_types=[
        pltpu.VMEM((HFULL + 1, HC), i32),
        pltpu.VMEM((HC,), f32),
        pltpu.VMEM((640,), f32),
        pltpu.VMEM_SHARED((DEG_S,), f32),
        pltpu.SemaphoreType.DMA,
    ],
)
def _sc_degree(dst_hbm, out_hbm, idx_v, val_v, z_v, deg_s, sem):
    cid = lax.axis_index("c")
    sid = lax.axis_index("s")
    wid = sid * NC + cid
    extra = wid < HEXTRA
    base = wid * HFULL + jnp.minimum(wid, HEXTRA)

    @pl.loop(0, HC // LANES)
    def _(i):
        val_v[pl.ds(i * LANES, LANES)] = jnp.full((LANES,), 1.0, f32)

    @pl.loop(0, 640 // LANES)
    def _(i):
        z_v[pl.ds(i * LANES, LANES)] = jnp.zeros((LANES,), f32)

    pltpu.sync_copy(z_v, deg_s.at[pl.ds(sid * 640, 640)])
    pltpu.sync_copy(dst_hbm.at[pl.ds(base, HFULL)], idx_v.at[pl.ds(0, HFULL)])

    @pl.when(extra)
    def _():
        pltpu.sync_copy(dst_hbm.at[pl.ds(base + HFULL, 1)],
                        idx_v.at[pl.ds(HFULL, 1)])

    plsc.subcore_barrier()

    descs = [
        pltpu.async_copy(val_v, deg_s.at[idx_v.at[ch]], sem, add=True)
        for ch in range(HFULL)
    ]

    @pl.when(extra)
    def _():
        pltpu.async_copy(val_v, deg_s.at[idx_v.at[HFULL]], sem, add=True)

    for d_ in descs:
        d_.wait()

    @pl.when(extra)
    def _():
        pltpu.make_async_copy(val_v, deg_s.at[idx_v.at[HFULL]], sem).wait()

    plsc.subcore_barrier()

    @pl.when(sid == 0)
    def _():
        pltpu.sync_copy(deg_s, out_hbm.at[cid])


def _make_sc_agg(h):
    @functools.partial(
        pl.kernel,
        out_type=jax.ShapeDtypeStruct((NC, NP, h), f32),
        mesh=_mesh,
        compiler_params=_sc_params,
        scratch_types=[
            pltpu.VMEM((CPT, C), i32),
            pltpu.VMEM((CPT, C), i32),
            pltpu.VMEM((NB * C, h), f32),
            pltpu.VMEM_SHARED((N, h), f32),
            [pltpu.SemaphoreType.DMA] * 4,
            [pltpu.SemaphoreType.DMA] * 4,
        ],
    )
    def agg(src_hbm, dst_hbm, rows_hbm, z_hbm, out_hbm,
            si_v, di_v, rows_v, acc_s, gsems, ssems):
        cid = lax.axis_index("c")
        sid = lax.axis_index("s")
        wid = sid * NC + cid

        pltpu.sync_copy(z_hbm, acc_s.at[pl.ds(sid * NPT, NPT)])
        pltpu.sync_copy(src_hbm.at[wid], si_v)
        pltpu.sync_copy(dst_hbm.at[wid], di_v)

        @pl.when(sid == 0)
        def _():
            pltpu.sync_copy(z_hbm.at[pl.ds(0, NP - N)],
                            out_hbm.at[cid, pl.ds(N, NP - N)])

        plsc.subcore_barrier()

        def gather(g):
            return pltpu.async_copy(rows_hbm.at[si_v.at[g]],
                                    rows_v.at[pl.ds((g % NB) * C, C)],
                                    gsems[g % 4])

        def scatter(g):
            return pltpu.async_copy(rows_v.at[pl.ds((g % NB) * C, C)],
                                    acc_s.at[di_v.at[g]],
                                    ssems[g % 4], add=True)

        gd = {g: gather(g) for g in range(LA)}
        sd = {}
        for g in range(CPT):
            gd.pop(g).wait()
            sd[g] = scatter(g)
            if g - SL in sd:
                sd.pop(g - SL).wait()
            if g + LA < CPT:
                gd[g + LA] = gather(g + LA)
        for g_ in sorted(sd):
            sd.pop(g_).wait()

        plsc.subcore_barrier()
        pltpu.sync_copy(acc_s.at[pl.ds(sid * NPT, NPT)],
                        out_hbm.at[cid, pl.ds(sid * NPT, NPT)])

    return agg


_sc_agg32 = _make_sc_agg(H1)
_sc_agg16 = _make_sc_agg(H2P)


def _tc_h1(x, w1):
    def body(x_ref, w_ref, o_ref):
        o_ref[pl.ds(0, N)] = jnp.dot(x_ref[...], w_ref[...],
                                     preferred_element_type=f32)
        o_ref[pl.ds(N, NP - N)] = jnp.zeros((NP - N, H1), f32)

    return pl.pallas_call(
        body, out_shape=jax.ShapeDtypeStruct((NP, H1), f32))(x, w1)


def _tc_scale(dpp, h):
    def body(dp_ref, h_ref, o_ref):
        deg = dp_ref[:, 0:1] + dp_ref[:, 1:2] + 1.0
        o_ref[...] = h_ref[...] * lax.rsqrt(deg)

    return pl.pallas_call(
        body, out_shape=jax.ShapeDtypeStruct((NP, H1), f32))(dpp, h)


def _tc_l2(dpp, qp, hp, b1r, w2p):
    def body(dp_ref, q_ref, hp_ref, b1_ref, w2_ref, o_ref):
        dinv = lax.rsqrt(dp_ref[:, 0:1] + dp_ref[:, 1:2] + 1.0)
        out1 = jnp.maximum(
            (q_ref[0] + q_ref[1] + hp_ref[...]) * dinv + b1_ref[...], 0.0)
        h2 = jnp.dot(out1, w2_ref[...], preferred_element_type=f32) * dinv
        rows = lax.broadcasted_iota(i32, (NP, 1), 0)
        o_ref[...] = jnp.where(rows < N, h2, 0.0)

    return pl.pallas_call(
        body, out_shape=jax.ShapeDtypeStruct((NP, H2P), f32))(
            dpp, qp, hp, b1r, w2p)


def _tc_out(dpp, rp, h2p, b2p):
    def body(dp_ref, r_ref, h2_ref, b2_ref, o_ref):
        dinv = lax.rsqrt(dp_ref[:, 0:1] + dp_ref[:, 1:2] + 1.0)
        full = (r_ref[0] + r_ref[1] + h2_ref[...]) * dinv + b2_ref[...]
        o_ref[...] = full[:N, :H2]

    return pl.pallas_call(
        body, out_shape=jax.ShapeDtypeStruct((N, H2), f32))(
            dpp, rp, h2p, b2p)


def kernel(x, edge_index, W1, b1, W2, b2):
    src2 = edge_index[0].reshape(NW, EPW_RAW)
    dst2 = edge_index[1].reshape(NW, EPW_RAW)
    pad = EPT - EPW_RAW
    pad_src = N + (jnp.arange(NW * pad, dtype=i32) % (NP - N)).reshape(NW, pad)
    pad_dst = (jnp.arange(NW * pad, dtype=i32) % N).reshape(NW, pad)
    src3 = jnp.concatenate([src2, pad_src], axis=1).reshape(NW, CPT, C)
    dst3 = jnp.concatenate([dst2, pad_dst], axis=1).reshape(NW, CPT, C)

    w2p = jnp.pad(W2, ((0, 0), (0, H2P - H2)))
    b1r = b1.reshape(1, H1)
    b2p = jnp.pad(b2, (0, H2P - H2)).reshape(1, H2P)

    z1 = jnp.zeros((NPT, H1), f32)
    z2 = jnp.zeros((NPT, H2P), f32)
    dstR = edge_index[1].reshape(HCT, HC)

    degp = _sc_degree(dstR)
    h = _tc_h1(x, W1)
    dpp = jnp.pad(degp[:, :N].T, ((0, NP - N), (0, 0)))
    hp = _tc_scale(dpp, h)
    qp = _sc_agg32(src3, dst3, hp, z1)
    h2p = _tc_l2(dpp, qp, hp, b1r, w2p)
    rp = _sc_agg16(src3, dst3, h2p, z2)
    return _tc_out(dpp, rp, h2p, b2p)


# speedup vs baseline: 1.0735x; 1.0735x over previous
import functools

import jax
import jax.numpy as jnp
from jax import lax
from jax.experimental import pallas as pl
from jax.experimental.pallas import tpu as pltpu
from jax.experimental.pallas import tpu_sc as plsc

N = 10000
E = 320000
D = 128
H1 = 32
H2 = 2
H2P = 16

NC = 2
NS = 16
NW = NC * NS
LANES = 16

C = 512
EPW_RAW = E // NW
CPT = 20
EPT = CPT * C
NB = 5
LA = 3
SL = 2

NP = 10016
NPT = N // NS

DEG_S = NS * 640
HC = 512
HCT = E // HC
HFULL = HCT // NW
HEXTRA = HCT - HFULL * NW

f32 = jnp.float32
i32 = jnp.int32

_mesh = plsc.VectorSubcoreMesh(core_axis_name="c", subcore_axis_name="s")
_sc_params = pltpu.CompilerParams(use_tc_tiling_on_sc=False)


@functools.partial(
    pl.kernel,
    out_type=jax.ShapeDtypeStruct((NC, DEG_S), f32),
    mesh=_mesh,
    compiler_params=_sc_params,
    scratch_types=[
        pltpu.VMEM((HFULL + 1, HC), i32),
        pltpu.VMEM((HC,), f32),
        pltpu.VMEM((640,), f32),
        pltpu.VMEM_SHARED((DEG_S,), f32),
        pltpu.SemaphoreType.DMA,
    ],
)
def _sc_degree(dst_hbm, out_hbm, idx_v, val_v, z_v, deg_s, sem):
    cid = lax.axis_index("c")
    sid = lax.axis_index("s")
    wid = sid * NC + cid
    extra = wid < HEXTRA
    base = wid * HFULL + jnp.minimum(wid, HEXTRA)

    @pl.loop(0, HC // LANES)
    def _(i):
        val_v[pl.ds(i * LANES, LANES)] = jnp.full((LANES,), 1.0, f32)

    @pl.loop(0, 640 // LANES)
    def _(i):
        z_v[pl.ds(i * LANES, LANES)] = jnp.zeros((LANES,), f32)

    pltpu.sync_copy(z_v, deg_s.at[pl.ds(sid * 640, 640)])
    pltpu.sync_copy(dst_hbm.at[pl.ds(base, HFULL)], idx_v.at[pl.ds(0, HFULL)])

    @pl.when(extra)
    def _():
        pltpu.sync_copy(dst_hbm.at[pl.ds(base + HFULL, 1)],
                        idx_v.at[pl.ds(HFULL, 1)])

    plsc.subcore_barrier()

    descs = [
        pltpu.async_copy(val_v, deg_s.at[idx_v.at[ch]], sem, add=True)
        for ch in range(HFULL)
    ]

    @pl.when(extra)
    def _():
        pltpu.async_copy(val_v, deg_s.at[idx_v.at[HFULL]], sem, add=True)

    for d_ in descs:
        d_.wait()

    @pl.when(extra)
    def _():
        pltpu.make_async_copy(val_v, deg_s.at[idx_v.at[HFULL]], sem).wait()

    plsc.subcore_barrier()

    @pl.when(sid == 0)
    def _():
        pltpu.sync_copy(deg_s, out_hbm.at[cid])


def _make_sc_agg(h):
    @functools.partial(
        pl.kernel,
        out_type=jax.ShapeDtypeStruct((NC, NP, h), f32),
        mesh=_mesh,
        compiler_params=_sc_params,
        scratch_types=[
            pltpu.VMEM((CPT, C), i32),
            pltpu.VMEM((CPT, C), i32),
            pltpu.VMEM((NB * C, h), f32),
            pltpu.VMEM_SHARED((N, h), f32),
            [pltpu.SemaphoreType.DMA] * 4,
            [pltpu.SemaphoreType.DMA] * 4,
        ],
    )
    def agg(src_hbm, dst_hbm, rows_hbm, z_hbm, out_hbm,
            si_v, di_v, rows_v, acc_s, gsems, ssems):
        cid = lax.axis_index("c")
        sid = lax.axis_index("s")
        wid = sid * NC + cid

        pltpu.sync_copy(z_hbm, acc_s.at[pl.ds(sid * NPT, NPT)])
        pltpu.sync_copy(src_hbm.at[wid], si_v)
        pltpu.sync_copy(dst_hbm.at[wid], di_v)

        @pl.when(sid == 0)
        def _():
            pltpu.sync_copy(z_hbm.at[pl.ds(0, NP - N)],
                            out_hbm.at[cid, pl.ds(N, NP - N)])

        plsc.subcore_barrier()

        def gather(g):
            return pltpu.async_copy(rows_hbm.at[si_v.at[g]],
                                    rows_v.at[pl.ds((g % NB) * C, C)],
                                    gsems[g % 4])

        def scatter(g):
            return pltpu.async_copy(rows_v.at[pl.ds((g % NB) * C, C)],
                                    acc_s.at[di_v.at[g]],
                                    ssems[g % 4], add=True)

        gd = {g: gather(g) for g in range(LA)}
        sd = {}
        for g in range(CPT):
            gd.pop(g).wait()
            sd[g] = scatter(g)
            if g - SL in sd:
                sd.pop(g - SL).wait()
            if g + LA < CPT:
                gd[g + LA] = gather(g + LA)
        for g_ in sorted(sd):
            sd.pop(g_).wait()

        plsc.subcore_barrier()
        pltpu.sync_copy(acc_s.at[pl.ds(sid * NPT, NPT)],
                        out_hbm.at[cid, pl.ds(sid * NPT, NPT)])

    return agg


_sc_agg32 = _make_sc_agg(H1)
_sc_agg16 = _make_sc_agg(H2P)


def _tc_h1(x, w1):
    def body(x_ref, w_ref, o_ref):
        o_ref[pl.ds(0, N)] = jnp.dot(x_ref[...], w_ref[...],
                                     preferred_element_type=f32)
        o_ref[pl.ds(N, NP - N)] = jnp.zeros((NP - N, H1), f32)

    return pl.pallas_call(
        body, out_shape=jax.ShapeDtypeStruct((NP, H1), f32))(x, w1)


def _tc_scale(dpp, h):
    def body(dp_ref, h_ref, o_ref):
        deg = dp_ref[:, 0:1] + dp_ref[:, 1:2] + 1.0
        o_ref[...] = h_ref[...] * lax.rsqrt(deg)

    return pl.pallas_call(
        body, out_shape=jax.ShapeDtypeStruct((NP, H1), f32))(dpp, h)


def _tc_l2(dpp, qp, hp, b1r, w2p):
    def body(dp_ref, q_ref, hp_ref, b1_ref, w2_ref, o_ref):
        dinv = lax.rsqrt(dp_ref[:, 0:1] + dp_ref[:, 1:2] + 1.0)
        out1 = jnp.maximum(
            (q_ref[0] + q_ref[1] + hp_ref[...]) * dinv + b1_ref[...], 0.0)
        h2 = jnp.dot(out1, w2_ref[...], preferred_element_type=f32) * dinv
        rows = lax.broadcasted_iota(i32, (NP, 1), 0)
        o_ref[...] = jnp.where(rows < N, h2, 0.0)

    return pl.pallas_call(
        body, out_shape=jax.ShapeDtypeStruct((NP, H2P), f32))(
            dpp, qp, hp, b1r, w2p)


def _tc_out(dpp, rp, h2p, b2p):
    def body(dp_ref, r_ref, h2_ref, b2_ref, o_ref):
        dinv = lax.rsqrt(dp_ref[:, 0:1] + dp_ref[:, 1:2] + 1.0)
        full = (r_ref[0] + r_ref[1] + h2_ref[...]) * dinv + b2_ref[...]
        o_ref[...] = full[:N, :H2]

    return pl.pallas_call(
        body, out_shape=jax.ShapeDtypeStruct((N, H2), f32))(
            dpp, rp, h2p, b2p)


def kernel(x, edge_index, W1, b1, W2, b2):
    src2 = edge_index[0].reshape(NW, EPW_RAW)
    dst2 = edge_index[1].reshape(NW, EPW_RAW)
    pad = EPT - EPW_RAW
    pad_src = N + (jnp.arange(NW * pad, dtype=i32) % (NP - N)).reshape(NW, pad)
    pad_dst = (jnp.arange(NW * pad, dtype=i32) % N).reshape(NW, pad)
    src3 = jnp.concatenate([src2, pad_src], axis=1).reshape(NW, CPT, C)
    dst3 = jnp.concatenate([dst2, pad_dst], axis=1).reshape(NW, CPT, C)

    w2p = jnp.pad(W2, ((0, 0), (0, H2P - H2)))
    b1r = b1.reshape(1, H1)
    b2p = jnp.pad(b2, (0, H2P - H2)).reshape(1, H2P)

    z1 = jnp.zeros((NPT, H1), f32)
    z2 = jnp.zeros((NPT, H2P), f32)
    dstR = edge_index[1].reshape(HCT, HC)

    degp = _sc_degree(dstR)
    h = _tc_h1(x, W1)
    dpp = jnp.pad(degp[:, :N].T, ((0, NP - N), (0, 0)))
    hp = _tc_scale(dpp, h)
    qp = _sc_agg32(src3, dst3, hp, z1)
    h2p = _tc_l2(dpp, qp, hp, b1r, w2p)
    rp = _sc_agg16(src3, dst3, h2p, z2)
    return _tc_out(dpp, rp, h2p, b2p)


# speedup vs baseline: 1.3192x; 1.2289x over previous
import functools

import jax
import jax.numpy as jnp
from jax import lax
from jax.experimental import pallas as pl
from jax.experimental.pallas import tpu as pltpu
from jax.experimental.pallas import tpu_sc as plsc

N = 10000
E = 320000
D = 128
H1 = 32
H2 = 2
H2P = 16

NC = 2
NS = 16
NW = NC * NS
LANES = 16

NB = 5
LA = 3
SL = 2

NPT = N // NS

DEG_S = NS * 640
HC = 512
HCT = E // HC
HFULL = HCT // NW
HEXTRA = HCT - HFULL * NW

f32 = jnp.float32
i32 = jnp.int32

_mesh = plsc.VectorSubcoreMesh(core_axis_name="c", subcore_axis_name="s")
_sc_params = pltpu.CompilerParams(use_tc_tiling_on_sc=False)


@functools.partial(
    pl.kernel,
    out_type=jax.ShapeDtypeStruct((NC, DEG_S), f32),
    mesh=_mesh,
    compiler_params=_sc_params,
    scratch_types=[
        pltpu.VMEM((HFULL + 1, HC), i32),
        pltpu.VMEM((HC,), f32),
        pltpu.VMEM((640,), f32),
        pltpu.VMEM_SHARED((DEG_S,), f32),
        pltpu.SemaphoreType.DMA,
    ],
)
def _sc_degree(dst_hbm, out_hbm, idx_v, val_v, z_v, deg_s, sem):
    cid = lax.axis_index("c")
    sid = lax.axis_index("s")
    wid = sid * NC + cid
    extra = wid < HEXTRA
    base = wid * HFULL + jnp.minimum(wid, HEXTRA)

    @pl.loop(0, HC // LANES)
    def _(i):
        val_v[pl.ds(i * LANES, LANES)] = jnp.full((LANES,), 1.0, f32)

    @pl.loop(0, 640 // LANES)
    def _(i):
        z_v[pl.ds(i * LANES, LANES)] = jnp.zeros((LANES,), f32)

    pltpu.sync_copy(z_v, deg_s.at[pl.ds(sid * 640, 640)])
    pltpu.sync_copy(dst_hbm.at[pl.ds(base, HFULL)], idx_v.at[pl.ds(0, HFULL)])

    @pl.when(extra)
    def _():
        pltpu.sync_copy(dst_hbm.at[pl.ds(base + HFULL, 1)],
                        idx_v.at[pl.ds(HFULL, 1)])

    plsc.subcore_barrier()

    descs = [
        pltpu.async_copy(val_v, deg_s.at[idx_v.at[ch]], sem, add=True)
        for ch in range(HFULL)
    ]

    @pl.when(extra)
    def _():
        pltpu.async_copy(val_v, deg_s.at[idx_v.at[HFULL]], sem, add=True)

    for d_ in descs:
        d_.wait()

    @pl.when(extra)
    def _():
        pltpu.make_async_copy(val_v, deg_s.at[idx_v.at[HFULL]], sem).wait()

    plsc.subcore_barrier()

    @pl.when(sid == 0)
    def _():
        pltpu.sync_copy(deg_s, out_hbm.at[cid])


def _make_sc_agg(h):
    @functools.partial(
        pl.kernel,
        out_type=jax.ShapeDtypeStruct((NC, N, h), f32),
        mesh=_mesh,
        compiler_params=_sc_params,
        scratch_types=[
            pltpu.VMEM((HFULL + 1, HC), i32),
            pltpu.VMEM((HFULL + 1, HC), i32),
            pltpu.VMEM((NB * HC, h), f32),
            pltpu.VMEM_SHARED((N, h), f32),
            [pltpu.SemaphoreType.DMA] * 4,
            [pltpu.SemaphoreType.DMA] * 4,
        ],
    )
    def agg(src_hbm, dst_hbm, rows_hbm, z_hbm, out_hbm,
            si_v, di_v, rows_v, acc_s, gsems, ssems):
        cid = lax.axis_index("c")
        sid = lax.axis_index("s")
        wid = sid * NC + cid
        extra = wid < HEXTRA
        base = wid * HFULL + jnp.minimum(wid, HEXTRA)

        pltpu.sync_copy(z_hbm, acc_s.at[pl.ds(sid * NPT, NPT)])
        pltpu.sync_copy(src_hbm.at[pl.ds(base, HFULL)],
                        si_v.at[pl.ds(0, HFULL)])
        pltpu.sync_copy(dst_hbm.at[pl.ds(base, HFULL)],
                        di_v.at[pl.ds(0, HFULL)])

        @pl.when(extra)
        def _():
            pltpu.sync_copy(src_hbm.at[pl.ds(base + HFULL, 1)],
                            si_v.at[pl.ds(HFULL, 1)])
            pltpu.sync_copy(dst_hbm.at[pl.ds(base + HFULL, 1)],
                            di_v.at[pl.ds(HFULL, 1)])

        plsc.subcore_barrier()

        def gather(g):
            return pltpu.async_copy(rows_hbm.at[si_v.at[g]],
                                    rows_v.at[pl.ds((g % NB) * HC, HC)],
                                    gsems[g % 4])

        def scatter(g):
            return pltpu.async_copy(rows_v.at[pl.ds((g % NB) * HC, HC)],
                                    acc_s.at[di_v.at[g]],
                                    ssems[g % 4], add=True)

        gd = {g: gather(g) for g in range(LA)}
        sd = {}
        for g in range(HFULL):
            gd.pop(g).wait()
            sd[g] = scatter(g)
            if g - SL in sd:
                sd.pop(g - SL).wait()
            if g + LA < HFULL:
                gd[g + LA] = gather(g + LA)
        for g_ in sorted(sd):
            sd.pop(g_).wait()

        @pl.when(extra)
        def _():
            gather(HFULL).wait()
            scatter(HFULL).wait()

        plsc.subcore_barrier()
        pltpu.sync_copy(acc_s.at[pl.ds(sid * NPT, NPT)],
                        out_hbm.at[cid, pl.ds(sid * NPT, NPT)])

    return agg


_sc_agg32 = _make_sc_agg(H1)
_sc_agg16 = _make_sc_agg(H2P)


def _tc_h1(x, w1):
    def body(x_ref, w_ref, o_ref):
        o_ref[...] = jnp.dot(x_ref[...], w_ref[...],
                             preferred_element_type=f32)

    return pl.pallas_call(
        body, out_shape=jax.ShapeDtypeStruct((N, H1), f32))(x, w1)


def _tc_scale(dpp, h):
    def body(dp_ref, h_ref, o_ref):
        deg = dp_ref[:, 0:1] + dp_ref[:, 1:2] + 1.0
        o_ref[...] = h_ref[...] * lax.rsqrt(deg)

    return pl.pallas_call(
        body, out_shape=jax.ShapeDtypeStruct((N, H1), f32))(dpp, h)


def _tc_l2(dpp, qp, hp, b1r, w2p):
    def body(dp_ref, q_ref, hp_ref, b1_ref, w2_ref, o_ref):
        dinv = lax.rsqrt(dp_ref[:, 0:1] + dp_ref[:, 1:2] + 1.0)
        out1 = jnp.maximum(
            (q_ref[0] + q_ref[1] + hp_ref[...]) * dinv + b1_ref[...], 0.0)
        o_ref[...] = jnp.dot(out1, w2_ref[...],
                             preferred_element_type=f32) * dinv

    return pl.pallas_call(
        body, out_shape=jax.ShapeDtypeStruct((N, H2P), f32))(
            dpp, qp, hp, b1r, w2p)


def _tc_out(dpp, rp, h2p, b2p):
    def body(dp_ref, r_ref, h2_ref, b2_ref, o_ref):
        dinv = lax.rsqrt(dp_ref[:, 0:1] + dp_ref[:, 1:2] + 1.0)
        full = (r_ref[0] + r_ref[1] + h2_ref[...]) * dinv + b2_ref[...]
        o_ref[...] = full[:, :H2]

    return pl.pallas_call(
        body, out_shape=jax.ShapeDtypeStruct((N, H2), f32))(
            dpp, rp, h2p, b2p)


def kernel(x, edge_index, W1, b1, W2, b2):
    srcR = edge_index[0].reshape(HCT, HC)
    dstR = edge_index[1].reshape(HCT, HC)

    w2p = jnp.pad(W2, ((0, 0), (0, H2P - H2)))
    b1r = b1.reshape(1, H1)
    b2p = jnp.pad(b2, (0, H2P - H2)).reshape(1, H2P)
    z1 = jnp.zeros((NPT, H1), f32)
    z2 = jnp.zeros((NPT, H2P), f32)

    degp = _sc_degree(dstR)
    h = _tc_h1(x, W1)
    dpp = degp[:, :N].T
    hp = _tc_scale(dpp, h)
    qp = _sc_agg32(srcR, dstR, hp, z1)
    h2p = _tc_l2(dpp, qp, hp, b1r, w2p)
    rp = _sc_agg16(srcR, dstR, h2p, z2)
    return _tc_out(dpp, rp, h2p, b2p)
